# baseline (device time: 57134 ns/iter reference)
import functools

import jax
import jax.numpy as jnp
from jax import lax
from jax.experimental import pallas as pl
from jax.experimental.pallas import tpu as pltpu

N_DEV = 8
N_ROUNDS = 3
N_LAYERS = 3


def kernel(x, Win0, Wout0, Win1, Wout1, Win2, Wout2):
    b, d_sh = x.shape
    _, h_dim = Win0.shape

    def body(x_ref, win0_ref, wout0_ref, win1_ref, wout1_ref, win2_ref,
             wout2_ref, out_ref, send_buf, recv_bufs, send_sems, recv_sems):
        my = lax.axis_index("i")

        barrier_sem = pltpu.get_barrier_semaphore()
        for r in range(N_ROUNDS):
            pl.semaphore_signal(
                barrier_sem, inc=1,
                device_id=(my ^ (1 << r),),
                device_id_type=pl.DeviceIdType.MESH,
            )
        pl.semaphore_wait(barrier_sem, N_ROUNDS)

        xv = x_ref[:, :].astype(jnp.bfloat16)
        wins = [win0_ref, win1_ref, win2_ref]
        wouts = [wout0_ref, wout1_ref, wout2_ref]

        for layer in range(N_LAYERS):
            partial = jnp.dot(
                xv, wins[layer][:, :].astype(jnp.bfloat16),
                preferred_element_type=jnp.float32,
            )

            acc = partial
            for r in range(N_ROUNDS):
                k = layer * N_ROUNDS + r
                partner = my ^ (1 << r)
                send_buf[:, :] = acc.astype(jnp.bfloat16)
                rdma = pltpu.make_async_remote_copy(
                    src_ref=send_buf,
                    dst_ref=recv_bufs.at[k],
                    send_sem=send_sems.at[k],
                    recv_sem=recv_sems.at[k],
                    device_id=(partner,),
                    device_id_type=pl.DeviceIdType.MESH,
                )
                rdma.start()
                rdma.wait()
                acc = acc + recv_bufs[k, :, :].astype(jnp.float32)

            h = jnp.maximum(acc, 0.0).astype(jnp.bfloat16)
            nxt = jnp.dot(
                h, wouts[layer][:, :].astype(jnp.bfloat16),
                preferred_element_type=jnp.float32,
            )
            if layer < N_LAYERS - 1:
                xv = nxt.astype(jnp.bfloat16)
            else:
                out_ref[:, :] = nxt

        @functools.partial(
            pl.run_scoped, second_barrier=pltpu.SemaphoreType.REGULAR
        )
        def _(second_barrier):
            for r in range(N_ROUNDS):
                pl.semaphore_signal(
                    second_barrier, inc=1,
                    device_id=(my ^ (1 << r),),
                    device_id_type=pl.DeviceIdType.MESH,
                )
            pl.semaphore_wait(second_barrier, N_ROUNDS)

    n_slots = N_LAYERS * N_ROUNDS
    return pl.pallas_call(
        body,
        out_shape=jax.ShapeDtypeStruct((b, d_sh), jnp.float32),
        in_specs=[pl.BlockSpec(memory_space=pltpu.VMEM)] * 7,
        out_specs=pl.BlockSpec(memory_space=pltpu.VMEM),
        scratch_shapes=[
            pltpu.VMEM((b, h_dim), jnp.bfloat16),
            pltpu.VMEM((n_slots, b, h_dim), jnp.bfloat16),
            pltpu.SemaphoreType.DMA((n_slots,)),
            pltpu.SemaphoreType.DMA((n_slots,)),
        ],
        compiler_params=pltpu.CompilerParams(collective_id=0),
    )(x, Win0, Wout0, Win1, Wout1, Win2, Wout2)


# device time: 55185 ns/iter; 1.0353x vs baseline; 1.0353x over previous
import functools

import jax
import jax.numpy as jnp
from jax import lax
from jax.experimental import pallas as pl
from jax.experimental.pallas import tpu as pltpu

N_DEV = 8
N_ROUNDS = 3
N_LAYERS = 3
MASKS = (1, 3, 4)


def kernel(x, Win0, Wout0, Win1, Wout1, Win2, Wout2):
    b, d_sh = x.shape
    _, h_dim = Win0.shape

    def body(x_ref, win0_ref, wout0_ref, win1_ref, wout1_ref, win2_ref,
             wout2_ref, out_ref, send_buf, recv_bufs, send_sems, recv_sems):
        my = lax.axis_index("i")

        barrier_sem = pltpu.get_barrier_semaphore()
        for m in MASKS:
            pl.semaphore_signal(
                barrier_sem, inc=1,
                device_id=(my ^ m,),
                device_id_type=pl.DeviceIdType.MESH,
            )
        pl.semaphore_wait(barrier_sem, N_ROUNDS)

        xv = x_ref[:, :].astype(jnp.bfloat16)
        wins = [win0_ref, win1_ref, win2_ref]
        wouts = [wout0_ref, wout1_ref, wout2_ref]

        for layer in range(N_LAYERS):
            partial = jnp.dot(
                xv, wins[layer][:, :].astype(jnp.bfloat16),
                preferred_element_type=jnp.float32,
            )

            acc = partial
            for r in range(N_ROUNDS):
                k = layer * N_ROUNDS + r
                partner = my ^ MASKS[r]
                send_buf[:, :] = acc.astype(jnp.bfloat16)
                rdma = pltpu.make_async_remote_copy(
                    src_ref=send_buf,
                    dst_ref=recv_bufs.at[k],
                    send_sem=send_sems.at[k],
                    recv_sem=recv_sems.at[k],
                    device_id=(partner,),
                    device_id_type=pl.DeviceIdType.MESH,
                )
                rdma.start()
                rdma.wait()
                acc = acc + recv_bufs[k, :, :].astype(jnp.float32)

            h = jnp.maximum(acc, 0.0).astype(jnp.bfloat16)
            nxt = jnp.dot(
                h, wouts[layer][:, :].astype(jnp.bfloat16),
                preferred_element_type=jnp.float32,
            )
            if layer < N_LAYERS - 1:
                xv = nxt.astype(jnp.bfloat16)
            else:
                out_ref[:, :] = nxt

        @functools.partial(
            pl.run_scoped, second_barrier=pltpu.SemaphoreType.REGULAR
        )
        def _(second_barrier):
            for m in MASKS:
                pl.semaphore_signal(
                    second_barrier, inc=1,
                    device_id=(my ^ m,),
                    device_id_type=pl.DeviceIdType.MESH,
                )
            pl.semaphore_wait(second_barrier, N_ROUNDS)

    n_slots = N_LAYERS * N_ROUNDS
    return pl.pallas_call(
        body,
        out_shape=jax.ShapeDtypeStruct((b, d_sh), jnp.float32),
        in_specs=[pl.BlockSpec(memory_space=pltpu.VMEM)] * 7,
        out_specs=pl.BlockSpec(memory_space=pltpu.VMEM),
        scratch_shapes=[
            pltpu.VMEM((b, h_dim), jnp.bfloat16),
            pltpu.VMEM((n_slots, b, h_dim), jnp.bfloat16),
            pltpu.SemaphoreType.DMA((n_slots,)),
            pltpu.SemaphoreType.DMA((n_slots,)),
        ],
        compiler_params=pltpu.CompilerParams(collective_id=0),
    )(x, Win0, Wout0, Win1, Wout1, Win2, Wout2)


# device time: 39505 ns/iter; 1.4462x vs baseline; 1.3969x over previous
import functools

import jax
import jax.numpy as jnp
from jax import lax
from jax.experimental import pallas as pl
from jax.experimental.pallas import tpu as pltpu

N_DEV = 8
N_PEERS = N_DEV - 1
N_LAYERS = 3


def kernel(x, Win0, Wout0, Win1, Wout1, Win2, Wout2):
    b, d_sh = x.shape
    _, h_dim = Win0.shape
    rows = b // N_DEV

    def body(x_ref, win0_ref, wout0_ref, win1_ref, wout1_ref, win2_ref,
             wout2_ref, out_ref, part_buf, hs_buf, h_full, rs_recv,
             rs_ssems, rs_rsems, ag_ssems, ag_rsems):
        my = lax.axis_index("i")

        barrier_sem = pltpu.get_barrier_semaphore()
        for t in range(1, N_DEV):
            pl.semaphore_signal(
                barrier_sem, inc=1,
                device_id=(my ^ t,),
                device_id_type=pl.DeviceIdType.MESH,
            )
        pl.semaphore_wait(barrier_sem, N_PEERS)

        xv = x_ref[:, :].astype(jnp.bfloat16)
        wins = [win0_ref, win1_ref, win2_ref]
        wouts = [wout0_ref, wout1_ref, wout2_ref]

        for layer in range(N_LAYERS):
            partial = jnp.dot(
                xv, wins[layer][:, :].astype(jnp.bfloat16),
                preferred_element_type=jnp.float32,
            )
            part_buf[:, :] = partial.astype(jnp.bfloat16)

            rs = []
            for t in range(1, N_DEV):
                partner = my ^ t
                k = layer * N_PEERS + (t - 1)
                r = pltpu.make_async_remote_copy(
                    src_ref=part_buf.at[pl.ds(partner * rows, rows), :],
                    dst_ref=rs_recv.at[k],
                    send_sem=rs_ssems.at[k],
                    recv_sem=rs_rsems.at[k],
                    device_id=(partner,),
                    device_id_type=pl.DeviceIdType.MESH,
                )
                r.start()
                rs.append(r)

            acc = part_buf[pl.ds(my * rows, rows), :].astype(jnp.float32)
            for t, r in zip(range(1, N_DEV), rs):
                r.wait()
                k = layer * N_PEERS + (t - 1)
                acc = acc + rs_recv[k, :, :].astype(jnp.float32)
            hs = jnp.maximum(acc, 0.0).astype(jnp.bfloat16)
            hs_buf[:, :] = hs

            ag = []
            for t in range(1, N_DEV):
                partner = my ^ t
                k = layer * N_PEERS + (t - 1)
                r = pltpu.make_async_remote_copy(
                    src_ref=hs_buf,
                    dst_ref=h_full.at[pl.ds(my * rows, rows), :],
                    send_sem=ag_ssems.at[k],
                    recv_sem=ag_rsems.at[k],
                    device_id=(partner,),
                    device_id_type=pl.DeviceIdType.MESH,
                )
                r.start()
                ag.append(r)
            h_full[pl.ds(my * rows, rows), :] = hs
            for r in ag:
                r.wait()

            nxt = jnp.dot(
                h_full[:, :], wouts[layer][:, :].astype(jnp.bfloat16),
                preferred_element_type=jnp.float32,
            )
            if layer < N_LAYERS - 1:
                xv = nxt.astype(jnp.bfloat16)
            else:
                out_ref[:, :] = nxt

        @functools.partial(
            pl.run_scoped, second_barrier=pltpu.SemaphoreType.REGULAR
        )
        def _(second_barrier):
            for t in range(1, N_DEV):
                pl.semaphore_signal(
                    second_barrier, inc=1,
                    device_id=(my ^ t,),
                    device_id_type=pl.DeviceIdType.MESH,
                )
            pl.semaphore_wait(second_barrier, N_PEERS)

    n_slots = N_LAYERS * N_PEERS
    return pl.pallas_call(
        body,
        out_shape=jax.ShapeDtypeStruct((b, d_sh), jnp.float32),
        in_specs=[pl.BlockSpec(memory_space=pltpu.VMEM)] * 7,
        out_specs=pl.BlockSpec(memory_space=pltpu.VMEM),
        scratch_shapes=[
            pltpu.VMEM((b, h_dim), jnp.bfloat16),
            pltpu.VMEM((rows, h_dim), jnp.bfloat16),
            pltpu.VMEM((b, h_dim), jnp.bfloat16),
            pltpu.VMEM((n_slots, rows, h_dim), jnp.bfloat16),
            pltpu.SemaphoreType.DMA((n_slots,)),
            pltpu.SemaphoreType.DMA((n_slots,)),
            pltpu.SemaphoreType.DMA((n_slots,)),
            pltpu.SemaphoreType.DMA((n_slots,)),
        ],
        compiler_params=pltpu.CompilerParams(collective_id=0),
    )(x, Win0, Wout0, Win1, Wout1, Win2, Wout2)


# device time: 39088 ns/iter; 1.4617x vs baseline; 1.0107x over previous
import functools

import jax
import jax.numpy as jnp
from jax import lax
from jax.experimental import pallas as pl
from jax.experimental.pallas import tpu as pltpu

N_DEV = 8
N_PEERS = N_DEV - 1
N_LAYERS = 3


def kernel(x, Win0, Wout0, Win1, Wout1, Win2, Wout2):
    b, d_sh = x.shape
    _, h_dim = Win0.shape
    rows = b // N_DEV

    def body(x_ref, win0_ref, wout0_ref, win1_ref, wout1_ref, win2_ref,
             wout2_ref, out_ref, part_buf, hs_buf, h_full, rs_recv,
             rs_ssems, rs_rsems, ag_ssems, ag_rsems):
        my = lax.axis_index("i")
        bf = jnp.bfloat16
        f32 = jnp.float32
        wins = [win0_ref, win1_ref, win2_ref]
        wouts = [wout0_ref, wout1_ref, wout2_ref]

        barrier_sem = pltpu.get_barrier_semaphore()
        for t in range(1, N_DEV):
            pl.semaphore_signal(
                barrier_sem, inc=1,
                device_id=(my ^ t,),
                device_id_type=pl.DeviceIdType.MESH,
            )
        pl.semaphore_wait(barrier_sem, N_PEERS)

        def rs_send(layer, t):
            partner = my ^ t
            k = layer * N_PEERS + (t - 1)
            r = pltpu.make_async_remote_copy(
                src_ref=part_buf.at[pl.ds(partner * rows, rows), :],
                dst_ref=rs_recv.at[k],
                send_sem=rs_ssems.at[k],
                recv_sem=rs_rsems.at[k],
                device_id=(partner,),
                device_id_type=pl.DeviceIdType.MESH,
            )
            r.start()
            return r

        def rs_finish(layer, rs):
            acc = part_buf[pl.ds(my * rows, rows), :].astype(f32)
            for t, r in zip(range(1, N_DEV), rs):
                r.wait()
                acc = acc + rs_recv[layer * N_PEERS + (t - 1), :, :].astype(f32)
            return jnp.maximum(acc, 0.0).astype(bf)

        def ag_start(layer, hs):
            hs_buf[:, :] = hs
            ag = []
            for t in range(1, N_DEV):
                k = layer * N_PEERS + (t - 1)
                r = pltpu.make_async_remote_copy(
                    src_ref=hs_buf,
                    dst_ref=h_full.at[pl.ds(my * rows, rows), :],
                    send_sem=ag_ssems.at[k],
                    recv_sem=ag_rsems.at[k],
                    device_id=(my ^ t,),
                    device_id_type=pl.DeviceIdType.MESH,
                )
                r.start()
                ag.append(r)
            h_full[pl.ds(my * rows, rows), :] = hs
            return ag

        xv = x_ref[:, :].astype(bf)
        part_buf[:, :] = jnp.dot(
            xv, win0_ref[:, :].astype(bf), preferred_element_type=f32
        ).astype(bf)
        rs = [rs_send(0, t) for t in range(1, N_DEV)]

        for layer in range(N_LAYERS):
            hs = rs_finish(layer, rs)
            ag = ag_start(layer, hs)

            if layer < N_LAYERS - 1:
                wout = wouts[layer][:, :].astype(bf)
                win_next = wins[layer + 1][:, :].astype(bf)
                y = jnp.dot(hs, wout, preferred_element_type=f32).astype(bf)
                p_own = jnp.dot(y, win_next, preferred_element_type=f32)
                rs = []
                for t, r in zip(range(1, N_DEV), ag):
                    r.wait()
                    blk = h_full[pl.ds((my ^ t) * rows, rows), :]
                    y = jnp.dot(blk, wout, preferred_element_type=f32).astype(bf)
                    p = jnp.dot(y, win_next, preferred_element_type=f32)
                    part_buf[pl.ds((my ^ t) * rows, rows), :] = p.astype(bf)
                    rs.append(rs_send(layer + 1, t))
                part_buf[pl.ds(my * rows, rows), :] = p_own.astype(bf)
            else:
                for r in ag:
                    r.wait()
                out_ref[:, :] = jnp.dot(
                    h_full[:, :], wouts[layer][:, :].astype(bf),
                    preferred_element_type=f32,
                )

        @functools.partial(
            pl.run_scoped, second_barrier=pltpu.SemaphoreType.REGULAR
        )
        def _(second_barrier):
            for t in range(1, N_DEV):
                pl.semaphore_signal(
                    second_barrier, inc=1,
                    device_id=(my ^ t,),
                    device_id_type=pl.DeviceIdType.MESH,
                )
            pl.semaphore_wait(second_barrier, N_PEERS)

    n_slots = N_LAYERS * N_PEERS
    return pl.pallas_call(
        body,
        out_shape=jax.ShapeDtypeStruct((b, d_sh), jnp.float32),
        in_specs=[pl.BlockSpec(memory_space=pltpu.VMEM)] * 7,
        out_specs=pl.BlockSpec(memory_space=pltpu.VMEM),
        scratch_shapes=[
            pltpu.VMEM((b, h_dim), jnp.bfloat16),
            pltpu.VMEM((rows, h_dim), jnp.bfloat16),
            pltpu.VMEM((b, h_dim), jnp.bfloat16),
            pltpu.VMEM((n_slots, rows, h_dim), jnp.bfloat16),
            pltpu.SemaphoreType.DMA((n_slots,)),
            pltpu.SemaphoreType.DMA((n_slots,)),
            pltpu.SemaphoreType.DMA((n_slots,)),
            pltpu.SemaphoreType.DMA((n_slots,)),
        ],
        compiler_params=pltpu.CompilerParams(collective_id=0),
    )(x, Win0, Wout0, Win1, Wout1, Win2, Wout2)
